# baseline (device time: 63433 ns/iter reference)
import jax
import jax.numpy as jnp
from jax import lax
from jax.experimental import pallas as pl
from jax.experimental.pallas import tpu as pltpu

N_DEV = 8


def kernel(x, w_mat):
    k_tot, _ = x.shape
    _, n = w_mat.shape
    m_chunk = k_tot // N_DEV

    def body(x_ref, w_ref, out_ref, stage_ref, comm_ref, send_sems, recv_sems):
        my = lax.axis_index("i")
        right = lax.rem(my + 1, N_DEV)

        def partial(c):
            xs = x_ref[pl.ds(c * m_chunk, m_chunk), :]
            return jnp.dot(xs, w_ref[:, :], preferred_element_type=jnp.float32)

        stage_ref[:, :] = partial(lax.rem(my - 1 + N_DEV, N_DEV))

        for s in range(N_DEV - 1):
            if s == 0:
                src = stage_ref
            else:
                comm_ref[s - 1, :, :] += partial(
                    lax.rem(my - 1 - s + 2 * N_DEV, N_DEV)
                )
                src = comm_ref.at[s - 1]
            rdma = pltpu.make_async_remote_copy(
                src_ref=src,
                dst_ref=comm_ref.at[s],
                send_sem=send_sems.at[s],
                recv_sem=recv_sems.at[s],
                device_id=(right,),
                device_id_type=pl.DeviceIdType.MESH,
            )
            rdma.start()
            rdma.wait()

        y = comm_ref[N_DEV - 2, :, :] + partial(my)
        out_ref[:, :] = y * jax.nn.sigmoid(y)

    return pl.pallas_call(
        body,
        out_shape=jax.ShapeDtypeStruct((m_chunk, n), jnp.float32),
        in_specs=[
            pl.BlockSpec(memory_space=pltpu.VMEM),
            pl.BlockSpec(memory_space=pltpu.VMEM),
        ],
        out_specs=pl.BlockSpec(memory_space=pltpu.VMEM),
        scratch_shapes=[
            pltpu.VMEM((m_chunk, n), jnp.float32),
            pltpu.VMEM((N_DEV - 1, m_chunk, n), jnp.float32),
            pltpu.SemaphoreType.DMA((N_DEV - 1,)),
            pltpu.SemaphoreType.DMA((N_DEV - 1,)),
        ],
    )(x, w_mat)


# device time: 41978 ns/iter; 1.5111x vs baseline; 1.5111x over previous
import jax
import jax.numpy as jnp
from jax import lax
from jax.experimental import pallas as pl
from jax.experimental.pallas import tpu as pltpu

N_DEV = 8


def kernel(x, w_mat):
    k_tot, _ = x.shape
    _, n = w_mat.shape
    m_chunk = k_tot // N_DEV

    def body(x_ref, w_ref, out_ref, send_buf, recv_buf, send_sems, recv_sems):
        my = lax.axis_index("i")

        def partial(c):
            xs = x_ref[pl.ds(c * m_chunk, m_chunk), :]
            return jnp.dot(xs, w_ref[:, :], preferred_element_type=jnp.float32)

        rdmas = []
        for j in range(N_DEV - 1):
            d = lax.rem(my - 1 - j + 2 * N_DEV, N_DEV)
            send_buf[j, :, :] = partial(d)
            rdma = pltpu.make_async_remote_copy(
                src_ref=send_buf.at[j],
                dst_ref=recv_buf.at[j],
                send_sem=send_sems.at[j],
                recv_sem=recv_sems.at[j],
                device_id=(d,),
                device_id_type=pl.DeviceIdType.MESH,
            )
            rdma.start()
            rdmas.append(rdma)

        acc = partial(my)
        for j in range(N_DEV - 1):
            rdmas[j].wait_recv()
            acc = acc + recv_buf[j, :, :]
        out_ref[:, :] = acc * jax.nn.sigmoid(acc)
        for j in range(N_DEV - 1):
            rdmas[j].wait_send()

    return pl.pallas_call(
        body,
        out_shape=jax.ShapeDtypeStruct((m_chunk, n), jnp.float32),
        in_specs=[
            pl.BlockSpec(memory_space=pltpu.VMEM),
            pl.BlockSpec(memory_space=pltpu.VMEM),
        ],
        out_specs=pl.BlockSpec(memory_space=pltpu.VMEM),
        scratch_shapes=[
            pltpu.VMEM((N_DEV - 1, m_chunk, n), jnp.float32),
            pltpu.VMEM((N_DEV - 1, m_chunk, n), jnp.float32),
            pltpu.SemaphoreType.DMA((N_DEV - 1,)),
            pltpu.SemaphoreType.DMA((N_DEV - 1,)),
        ],
    )(x, w_mat)


# device time: 39768 ns/iter; 1.5951x vs baseline; 1.0556x over previous
import jax
import jax.numpy as jnp
from jax import lax
from jax.experimental import pallas as pl
from jax.experimental.pallas import tpu as pltpu

N_DEV = 8
ROTS = ((0, 1, 2), (1, 2, 0), (2, 0, 1))
SLOTS = {0: (0, 1, 2), 1: (3, 4), 2: (5, 6), 4: (7, 8),
         3: (9,), 5: (10,), 6: (11,)}
N_RECV = 12


def kernel(x, w_mat):
    k_tot, _ = x.shape
    _, n = w_mat.shape
    m_chunk = k_tot // N_DEV

    def body(x_ref, w_ref, out_ref, send_buf, recv_buf, send_sems, recv_sems):
        my = lax.axis_index("i")
        cz = my // 4
        q = my % 4
        g = q ^ (q // 2)
        cx = g % 2
        cy = g // 2

        def pos_of(px, py, pz):
            gg = px + 2 * py
            return (gg ^ (gg // 2)) + 4 * pz

        d_pos, rho, nbr_pos = {}, {}, {}
        for e in range(1, 8):
            ex, ey, ez = e & 1, (e >> 1) & 1, (e >> 2) & 1
            dx, dy, dz = cx ^ ex, cy ^ ey, cz ^ ez
            d_pos[e] = pos_of(dx, dy, dz)
            rho[e] = dy * (1 + dx)
        for m in range(3):
            b = 1 << m
            nbr_pos[m] = pos_of(cx ^ (b & 1), cy ^ ((b >> 1) & 1),
                                cz ^ ((b >> 2) & 1))

        def partial(c):
            xs = x_ref[pl.ds(c * m_chunk, m_chunk), :]
            return jnp.dot(xs, w_ref[:, :], preferred_element_type=jnp.float32)

        def do_send(i, dst_slot, dst):
            pltpu.make_async_remote_copy(
                src_ref=send_buf.at[i], dst_ref=recv_buf.at[dst_slot],
                send_sem=send_sems.at[i], recv_sem=recv_sems.at[dst_slot],
                device_id=(dst,), device_id_type=pl.DeviceIdType.MESH,
            ).start()

        def recv_wait(slot):
            pltpu.make_async_remote_copy(
                src_ref=recv_buf.at[slot], dst_ref=recv_buf.at[slot],
                send_sem=send_sems.at[0], recv_sem=recv_sems.at[slot],
                device_id=(my,), device_id_type=pl.DeviceIdType.MESH,
            ).wait_recv()

        def send_wait(i):
            pltpu.make_async_remote_copy(
                src_ref=send_buf.at[i], dst_ref=send_buf.at[i],
                send_sem=send_sems.at[i], recv_sem=recv_sems.at[0],
                device_id=(my,), device_id_type=pl.DeviceIdType.MESH,
            ).wait_send()

        rounds = {0: [], 1: [], 2: []}
        for e in range(1, 8):
            for k in range(3):
                order = ROTS[k]
                pos = next(i for i, m in enumerate(order) if (e >> m) & 1)
                rounds[pos].append((e, k, order[pos]))

        for e in range(1, 8):
            send_buf[e - 1, :, :] = partial(d_pos[e])
        own = partial(my)

        for pos in (0, 1, 2):
            for e, k, m in rounds[pos]:
                dst_slot = SLOTS[e ^ (1 << m)][pos]

                @pl.when(rho[e] == k)
                def _(e=e, k=k, m=m, pos=pos, dst_slot=dst_slot):
                    for lp in range(pos):
                        s = SLOTS[e][lp]
                        recv_wait(s)
                        send_buf[e - 1, :, :] += recv_buf[s, :, :]
                    do_send(e - 1, dst_slot, nbr_pos[m])

        recv_wait(0)
        recv_wait(1)
        recv_wait(2)
        y = own + recv_buf[0] + recv_buf[1] + recv_buf[2]
        out_ref[:, :] = y * jax.nn.sigmoid(y)
        for i in range(7):
            send_wait(i)

    return pl.pallas_call(
        body,
        out_shape=jax.ShapeDtypeStruct((m_chunk, n), jnp.float32),
        in_specs=[
            pl.BlockSpec(memory_space=pltpu.VMEM),
            pl.BlockSpec(memory_space=pltpu.VMEM),
        ],
        out_specs=pl.BlockSpec(memory_space=pltpu.VMEM),
        scratch_shapes=[
            pltpu.VMEM((7, m_chunk, n), jnp.float32),
            pltpu.VMEM((N_RECV, m_chunk, n), jnp.float32),
            pltpu.SemaphoreType.DMA((7,)),
            pltpu.SemaphoreType.DMA((N_RECV,)),
        ],
    )(x, w_mat)


# device time: 28169 ns/iter; 2.2519x vs baseline; 1.4118x over previous
import jax
import jax.numpy as jnp
from jax import lax
from jax.experimental import pallas as pl
from jax.experimental.pallas import tpu as pltpu

N_DEV = 8
ROTS = ((0, 1, 2), (1, 2, 0), (2, 0, 1))
SLOTS = {0: (0, 1, 2), 1: (3, 4), 2: (5, 6), 4: (7, 8),
         3: (9,), 5: (10,), 6: (11,)}
N_RECV = 12


def kernel(x, w_mat):
    k_tot, _ = x.shape
    _, n = w_mat.shape
    m_chunk = k_tot // N_DEV

    def body(x_ref, w_ref, out_ref, send_buf, recv_buf, send_sems, recv_sems):
        my = lax.axis_index("i")
        cz = my // 4
        q = my % 4
        g = q ^ (q // 2)
        cx = g % 2
        cy = g // 2

        def pos_of(px, py, pz):
            gg = px + 2 * py
            return (gg ^ (gg // 2)) + 4 * pz

        d_pos, rho, nbr_pos = {}, {}, {}
        for e in range(1, 8):
            ex, ey, ez = e & 1, (e >> 1) & 1, (e >> 2) & 1
            dx, dy, dz = cx ^ ex, cy ^ ey, cz ^ ez
            d_pos[e] = pos_of(dx, dy, dz)
            rho[e] = dy * (1 + dx)
        for m in range(3):
            b = 1 << m
            nbr_pos[m] = pos_of(cx ^ (b & 1), cy ^ ((b >> 1) & 1),
                                cz ^ ((b >> 2) & 1))

        def partial(c):
            xs = x_ref[pl.ds(c * m_chunk, m_chunk), :]
            return jnp.dot(xs, w_ref[:, :], preferred_element_type=jnp.float32)

        def do_send(i, dst_slot, dst):
            pltpu.make_async_remote_copy(
                src_ref=send_buf.at[i], dst_ref=recv_buf.at[dst_slot],
                send_sem=send_sems.at[i], recv_sem=recv_sems.at[dst_slot],
                device_id=(dst,), device_id_type=pl.DeviceIdType.MESH,
            ).start()

        def recv_wait(slot):
            pltpu.make_async_remote_copy(
                src_ref=recv_buf.at[slot], dst_ref=recv_buf.at[slot],
                send_sem=send_sems.at[0], recv_sem=recv_sems.at[slot],
                device_id=(my,), device_id_type=pl.DeviceIdType.MESH,
            ).wait_recv()

        def send_wait(i):
            pltpu.make_async_remote_copy(
                src_ref=send_buf.at[i], dst_ref=send_buf.at[i],
                send_sem=send_sems.at[i], recv_sem=recv_sems.at[0],
                device_id=(my,), device_id_type=pl.DeviceIdType.MESH,
            ).wait_send()

        rounds = {0: [], 1: [], 2: []}
        for e in range(1, 8):
            for k in range(3):
                order = ROTS[k]
                pos = next(i for i, m in enumerate(order) if (e >> m) & 1)
                rounds[pos].append((e, k, order[pos]))

        for e in range(1, 8):
            send_buf[e - 1, :, :] = partial(d_pos[e]).astype(jnp.bfloat16)
        own = partial(my)

        for pos in (0, 1, 2):
            for e, k, m in rounds[pos]:
                dst_slot = SLOTS[e ^ (1 << m)][pos]

                @pl.when(rho[e] == k)
                def _(e=e, k=k, m=m, pos=pos, dst_slot=dst_slot):
                    for lp in range(pos):
                        s = SLOTS[e][lp]
                        recv_wait(s)
                        send_buf[e - 1, :, :] += recv_buf[s, :, :]
                    do_send(e - 1, dst_slot, nbr_pos[m])

        recv_wait(0)
        recv_wait(1)
        recv_wait(2)
        y = own + (recv_buf[0].astype(jnp.float32)
                   + recv_buf[1].astype(jnp.float32)
                   + recv_buf[2].astype(jnp.float32))
        out_ref[:, :] = y * jax.nn.sigmoid(y)
        for i in range(7):
            send_wait(i)

    return pl.pallas_call(
        body,
        out_shape=jax.ShapeDtypeStruct((m_chunk, n), jnp.float32),
        in_specs=[
            pl.BlockSpec(memory_space=pltpu.VMEM),
            pl.BlockSpec(memory_space=pltpu.VMEM),
        ],
        out_specs=pl.BlockSpec(memory_space=pltpu.VMEM),
        scratch_shapes=[
            pltpu.VMEM((7, m_chunk, n), jnp.bfloat16),
            pltpu.VMEM((N_RECV, m_chunk, n), jnp.bfloat16),
            pltpu.SemaphoreType.DMA((7,)),
            pltpu.SemaphoreType.DMA((N_RECV,)),
        ],
    )(x, w_mat)
